# baseline (speedup 1.0000x reference)
import functools

import jax
import jax.numpy as jnp
from jax import lax
from jax.experimental import pallas as pl
from jax.experimental.pallas import tpu as pltpu
from jax.experimental.pallas import tpu_sc as plsc

_B, _N, _S, _D = 8, 2048, 64, 128

_NC, _NS = 2, 16
_NW = _NC * _NS
_CH = 8


def _make_sc_body(nb, boff):
  rpw = (nb * _N) // _NW
  cpw = rpw // _CH
  wpb = _NW // nb
  ncolw = (nb * _S) // 16

  def body(dm_ref, idx_ref, rowg_ref, colg_ref,
           idx_v, idxc_v, cid_v, inbuf0, inbuf1, outbuf, colbuf,
           sem0, sem1, semc):
    c = lax.axis_index("c")
    s = lax.axis_index("s")
    wid = s * _NC + c
    b = wid // wpb
    row0 = boff * _N + wid * rpw

    pltpu.sync_copy(idx_ref.at[pl.ds((boff + b) * _S, _S)], idx_v)
    ivs = [idx_v[pl.ds(k * 16, 16)] for k in range(4)]

    @pl.when(wid < ncolw)
    def _():
      bc = wid // (_S // 16)
      pltpu.sync_copy(idx_ref.at[pl.ds((boff + bc) * _S, _S)], idxc_v)
      cid_v[...] = idxc_v[pl.ds((wid % (_S // 16)) * 16, 16)] + (
          (boff + bc) * _N)
      pltpu.async_copy(dm_ref.at[cid_v], colbuf, semc)

    bufs = (inbuf0, inbuf1)
    sems = (sem0, sem1)

    def _start(g, t):
      pltpu.async_copy(dm_ref.at[pl.ds(row0 + g * _CH, _CH)], bufs[t],
                       sems[t])

    def _wait(g, t):
      pltpu.make_async_copy(dm_ref.at[pl.ds(row0 + g * _CH, _CH)],
                            bufs[t], sems[t]).wait()

    _start(0, 0)

    def chunk2(gg, carry):
      for t in range(2):
        g = gg * 2 + t

        @pl.when(g + 1 < cpw)
        def _():
          _start(g + 1, 1 - t)

        _wait(g, t)
        for r in range(_CH):
          rvec = jnp.full((16,), r, jnp.int32)
          for k in range(4):
            outbuf[g * _CH + r, pl.ds(k * 16, 16)] = (
                plsc.load_gather(bufs[t], [rvec, ivs[k]]))
      return carry

    lax.fori_loop(0, cpw // 2, chunk2, 0)
    pltpu.sync_copy(outbuf, rowg_ref.at[pl.ds(wid * rpw, rpw)])

    @pl.when(wid < ncolw)
    def _():
      pltpu.make_async_copy(dm_ref.at[cid_v], colbuf, semc).wait()
      pltpu.sync_copy(colbuf, colg_ref.at[pl.ds(wid * 16, 16)])

  return body


def _sc_gather(dm2, idxflat, nb, boff):
  mesh = plsc.VectorSubcoreMesh(core_axis_name="c", subcore_axis_name="s",
                                num_cores=_NC, num_subcores=_NS)
  rpw = (nb * _N) // _NW
  f = pl.kernel(
      _make_sc_body(nb, boff),
      out_type=[
          jax.ShapeDtypeStruct((nb * _N, _S), jnp.float32),
          jax.ShapeDtypeStruct((nb * _S, _N), jnp.float32),
      ],
      mesh=mesh,
      scratch_types=[
          pltpu.VMEM((_S,), jnp.int32),
          pltpu.VMEM((_S,), jnp.int32),
          pltpu.VMEM((16,), jnp.int32),
          pltpu.VMEM((_CH, _N), jnp.float32),
          pltpu.VMEM((_CH, _N), jnp.float32),
          pltpu.VMEM((rpw, _S), jnp.float32),
          pltpu.VMEM((16, _N), jnp.float32),
          pltpu.SemaphoreType.DMA,
          pltpu.SemaphoreType.DMA,
          pltpu.SemaphoreType.DMA,
      ],
      compiler_params=pltpu.CompilerParams(needs_layout_passes=False),
      cost_estimate=pl.CostEstimate(
          flops=0,
          bytes_accessed=nb * _N * _N * 4 + nb * (_N + _S) * _S * 8,
          transcendentals=0,
      ),
  )
  return f(dm2, idxflat)


_RB = 2048
_PREC = lax.Precision.HIGHEST


def _xor_perm(j):
  r = lax.broadcasted_iota(jnp.int32, (_S, _S), 0)
  c = lax.broadcasted_iota(jnp.int32, (_S, _S), 1)
  return ((r ^ j) == c).astype(jnp.float32)


def _bitonic_sort_lanes(x, use_mxu):
  lane = lax.broadcasted_iota(jnp.int32, (1, _S), 1)
  k = 2
  while k <= _S:
    j = k // 2
    while j >= 1:
      lower = (lane & j) == 0
      if use_mxu:
        xp = lax.dot_general(x, _xor_perm(j), (((1,), (0,)), ((), ())),
                             preferred_element_type=jnp.float32)
      else:
        xp = jnp.where(lower,
                       pltpu.roll(x, _S - j, 1),
                       pltpu.roll(x, j, 1))
      take_min = lower == ((lane & k) == 0)
      x = jnp.where(take_min, jnp.minimum(x, xp), jnp.maximum(x, xp))
      j //= 2
    k *= 2
  return x


def _tc_body(rowg_ref, colg_ref, wr_ref, br_ref, wc_ref, bc_ref,
             row_out_ref, col_out_ref):
  rs = _bitonic_sort_lanes(rowg_ref[0], use_mxu=True)
  remb = lax.dot_general(rs, wr_ref[...], (((1,), (1,)), ((), ())),
                         preferred_element_type=jnp.float32,
                         precision=_PREC)
  row_out_ref[0] = remb + br_ref[...]

  cg = jnp.transpose(colg_ref[0], (1, 0))
  cs = _bitonic_sort_lanes(cg, use_mxu=True)
  cemb = lax.dot_general(cs, wc_ref[...], (((1,), (1,)), ((), ())),
                         preferred_element_type=jnp.float32,
                         precision=_PREC)
  col_out_ref[0] = cemb + bc_ref[...]


def _tc_sort_embed(rowg3, colg3, Wr, br2, Wc, bc2, Bv):
  grid = (Bv, _N // _RB)
  return pl.pallas_call(
      _tc_body,
      grid=grid,
      in_specs=[
          pl.BlockSpec((1, _RB, _S), lambda b, i: (b, i, 0)),
          pl.BlockSpec((1, _S, _RB), lambda b, i: (b, 0, i)),
          pl.BlockSpec((_D, _S), lambda b, i: (0, 0)),
          pl.BlockSpec((1, _D), lambda b, i: (0, 0)),
          pl.BlockSpec((_D, _S), lambda b, i: (0, 0)),
          pl.BlockSpec((1, _D), lambda b, i: (0, 0)),
      ],
      out_specs=[
          pl.BlockSpec((1, _RB, _D), lambda b, i: (b, i, 0)),
          pl.BlockSpec((1, _RB, _D), lambda b, i: (b, i, 0)),
      ],
      out_shape=[
          jax.ShapeDtypeStruct((Bv, _N, _D), jnp.float32),
          jax.ShapeDtypeStruct((Bv, _N, _D), jnp.float32),
      ],
      compiler_params=pltpu.CompilerParams(
          dimension_semantics=("arbitrary", "arbitrary"),
      ),
      cost_estimate=pl.CostEstimate(
          flops=2 * Bv * _N * _S * (21 * _S + 2 * _D),
          bytes_accessed=Bv * _N * (_S * 8 + _D * 8) * 4,
          transcendentals=0,
      ),
  )(rowg3, colg3, Wr, br2, Wc, bc2)


def kernel(distance_matrix, Wr, br, Wc, bc, phase):
  Bv = distance_matrix.shape[0]
  ikey = jax.random.key(42)
  ri = jax.random.randint(ikey, (8, 1, _S), 0, _N)
  idx = jnp.broadcast_to(ri[:, None, :, :], (8, Bv // 8, 1, _S))
  idxflat = idx.reshape(Bv * _S).astype(jnp.int32)

  dm2 = distance_matrix.reshape(Bv * _N, _N)
  br2 = br.reshape(1, _D)
  bc2 = bc.reshape(1, _D)

  nb = Bv // 4
  parts = []
  for boff in range(0, Bv, nb):
    rowg, colg = _sc_gather(dm2, idxflat, nb, boff)
    parts.append((rowg, colg))
  outs = []
  for rowg, colg in parts:
    outs.append(_tc_sort_embed(rowg.reshape(nb, _N, _S),
                               colg.reshape(nb, _S, _N),
                               Wr, br2, Wc, bc2, nb))
  row_emb = jnp.concatenate([o[0] for o in outs], axis=0)
  col_emb = jnp.concatenate([o[1] for o in outs], axis=0)
  return (row_emb, col_emb)


# speedup vs baseline: 1.0616x; 1.0616x over previous
import functools

import jax
import jax.numpy as jnp
from jax import lax
from jax.experimental import pallas as pl
from jax.experimental.pallas import tpu as pltpu
from jax.experimental.pallas import tpu_sc as plsc

_B, _N, _S, _D = 8, 2048, 64, 128

_NC, _NS = 2, 16
_NW = _NC * _NS
_CH = 4


def _make_sc_body(nb, boff):
  rpw = (nb * _N) // _NW
  cpw = rpw // _CH
  wpb = _NW // nb
  ncolw = (nb * _S) // 16

  def body(dm_ref, idx_ref, rowg_ref, colg_ref,
           idx_v, idxc_v, cid_v, inbuf0, inbuf1, inbuf2, inbuf3,
           outbuf, colbuf, sem0, sem1, sem2, sem3, semc):
    c = lax.axis_index("c")
    s = lax.axis_index("s")
    wid = s * _NC + c
    b = wid // wpb
    row0 = boff * _N + wid * rpw

    pltpu.sync_copy(idx_ref.at[pl.ds((boff + b) * _S, _S)], idx_v)
    ivs = [idx_v[pl.ds(k * 16, 16)] for k in range(4)]

    @pl.when(wid < ncolw)
    def _():
      bc = wid // (_S // 16)
      pltpu.sync_copy(idx_ref.at[pl.ds((boff + bc) * _S, _S)], idxc_v)
      cid_v[...] = idxc_v[pl.ds((wid % (_S // 16)) * 16, 16)] + (
          (boff + bc) * _N)
      pltpu.async_copy(dm_ref.at[cid_v], colbuf, semc)

    bufs = (inbuf0, inbuf1, inbuf2, inbuf3)
    sems = (sem0, sem1, sem2, sem3)
    nbuf = 4

    def _start(g, t):
      pltpu.async_copy(dm_ref.at[pl.ds(row0 + g * _CH, _CH)], bufs[t],
                       sems[t])

    def _wait(g, t):
      pltpu.make_async_copy(dm_ref.at[pl.ds(row0 + g * _CH, _CH)],
                            bufs[t], sems[t]).wait()

    for g in range(nbuf - 1):
      _start(g, g)

    def chunk4(gg, carry):
      for t in range(nbuf):
        g = gg * nbuf + t

        @pl.when(g + nbuf - 1 < cpw)
        def _():
          _start(g + nbuf - 1, (t + nbuf - 1) % nbuf)

        _wait(g, t)
        for r in range(_CH):
          rvec = jnp.full((16,), r, jnp.int32)
          for k in range(4):
            outbuf[g * _CH + r, pl.ds(k * 16, 16)] = (
                plsc.load_gather(bufs[t], [rvec, ivs[k]]))
      return carry

    lax.fori_loop(0, cpw // nbuf, chunk4, 0)
    pltpu.sync_copy(outbuf, rowg_ref.at[pl.ds(wid * rpw, rpw)])

    @pl.when(wid < ncolw)
    def _():
      pltpu.make_async_copy(dm_ref.at[cid_v], colbuf, semc).wait()
      pltpu.sync_copy(colbuf, colg_ref.at[pl.ds(wid * 16, 16)])

  return body


def _sc_gather(dm2, idxflat, nb, boff):
  mesh = plsc.VectorSubcoreMesh(core_axis_name="c", subcore_axis_name="s",
                                num_cores=_NC, num_subcores=_NS)
  rpw = (nb * _N) // _NW
  f = pl.kernel(
      _make_sc_body(nb, boff),
      out_type=[
          jax.ShapeDtypeStruct((nb * _N, _S), jnp.float32),
          jax.ShapeDtypeStruct((nb * _S, _N), jnp.float32),
      ],
      mesh=mesh,
      scratch_types=[
          pltpu.VMEM((_S,), jnp.int32),
          pltpu.VMEM((_S,), jnp.int32),
          pltpu.VMEM((16,), jnp.int32),
          pltpu.VMEM((_CH, _N), jnp.float32),
          pltpu.VMEM((_CH, _N), jnp.float32),
          pltpu.VMEM((_CH, _N), jnp.float32),
          pltpu.VMEM((_CH, _N), jnp.float32),
          pltpu.VMEM((rpw, _S), jnp.float32),
          pltpu.VMEM((16, _N), jnp.float32),
          pltpu.SemaphoreType.DMA,
          pltpu.SemaphoreType.DMA,
          pltpu.SemaphoreType.DMA,
          pltpu.SemaphoreType.DMA,
          pltpu.SemaphoreType.DMA,
      ],
      compiler_params=pltpu.CompilerParams(needs_layout_passes=False),
      cost_estimate=pl.CostEstimate(
          flops=0,
          bytes_accessed=nb * _N * _N * 4 + nb * (_N + _S) * _S * 8,
          transcendentals=0,
      ),
  )
  return f(dm2, idxflat)


_RB = 2048
_PREC = lax.Precision.HIGHEST


def _xor_perm(j):
  r = lax.broadcasted_iota(jnp.int32, (_S, _S), 0)
  c = lax.broadcasted_iota(jnp.int32, (_S, _S), 1)
  return ((r ^ j) == c).astype(jnp.float32)


def _bitonic_sort_lanes(x, use_mxu):
  lane = lax.broadcasted_iota(jnp.int32, (1, _S), 1)
  k = 2
  while k <= _S:
    j = k // 2
    while j >= 1:
      lower = (lane & j) == 0
      if use_mxu:
        xp = lax.dot_general(x, _xor_perm(j), (((1,), (0,)), ((), ())),
                             preferred_element_type=jnp.float32)
      else:
        xp = jnp.where(lower,
                       pltpu.roll(x, _S - j, 1),
                       pltpu.roll(x, j, 1))
      take_min = lower == ((lane & k) == 0)
      x = jnp.where(take_min, jnp.minimum(x, xp), jnp.maximum(x, xp))
      j //= 2
    k *= 2
  return x


def _tc_body(rowg_ref, colg_ref, wr_ref, br_ref, wc_ref, bc_ref,
             row_out_ref, col_out_ref):
  rs = _bitonic_sort_lanes(rowg_ref[0], use_mxu=True)
  remb = lax.dot_general(rs, wr_ref[...], (((1,), (1,)), ((), ())),
                         preferred_element_type=jnp.float32,
                         precision=_PREC)
  row_out_ref[0] = remb + br_ref[...]

  cg = jnp.transpose(colg_ref[0], (1, 0))
  cs = _bitonic_sort_lanes(cg, use_mxu=True)
  cemb = lax.dot_general(cs, wc_ref[...], (((1,), (1,)), ((), ())),
                         preferred_element_type=jnp.float32,
                         precision=_PREC)
  col_out_ref[0] = cemb + bc_ref[...]


def _tc_sort_embed(rowg3, colg3, Wr, br2, Wc, bc2, Bv):
  grid = (Bv, _N // _RB)
  return pl.pallas_call(
      _tc_body,
      grid=grid,
      in_specs=[
          pl.BlockSpec((1, _RB, _S), lambda b, i: (b, i, 0)),
          pl.BlockSpec((1, _S, _RB), lambda b, i: (b, 0, i)),
          pl.BlockSpec((_D, _S), lambda b, i: (0, 0)),
          pl.BlockSpec((1, _D), lambda b, i: (0, 0)),
          pl.BlockSpec((_D, _S), lambda b, i: (0, 0)),
          pl.BlockSpec((1, _D), lambda b, i: (0, 0)),
      ],
      out_specs=[
          pl.BlockSpec((1, _RB, _D), lambda b, i: (b, i, 0)),
          pl.BlockSpec((1, _RB, _D), lambda b, i: (b, i, 0)),
      ],
      out_shape=[
          jax.ShapeDtypeStruct((Bv, _N, _D), jnp.float32),
          jax.ShapeDtypeStruct((Bv, _N, _D), jnp.float32),
      ],
      compiler_params=pltpu.CompilerParams(
          dimension_semantics=("arbitrary", "arbitrary"),
      ),
      cost_estimate=pl.CostEstimate(
          flops=2 * Bv * _N * _S * (21 * _S + 2 * _D),
          bytes_accessed=Bv * _N * (_S * 8 + _D * 8) * 4,
          transcendentals=0,
      ),
  )(rowg3, colg3, Wr, br2, Wc, bc2)


def kernel(distance_matrix, Wr, br, Wc, bc, phase):
  Bv = distance_matrix.shape[0]
  ikey = jax.random.key(42)
  ri = jax.random.randint(ikey, (8, 1, _S), 0, _N)
  idx = jnp.broadcast_to(ri[:, None, :, :], (8, Bv // 8, 1, _S))
  idxflat = idx.reshape(Bv * _S).astype(jnp.int32)

  dm2 = distance_matrix.reshape(Bv * _N, _N)
  br2 = br.reshape(1, _D)
  bc2 = bc.reshape(1, _D)

  nb = Bv // 4
  parts = []
  for boff in range(0, Bv, nb):
    rowg, colg = _sc_gather(dm2, idxflat, nb, boff)
    parts.append((rowg, colg))
  outs = []
  for rowg, colg in parts:
    outs.append(_tc_sort_embed(rowg.reshape(nb, _N, _S),
                               colg.reshape(nb, _S, _N),
                               Wr, br2, Wc, bc2, nb))
  row_emb = jnp.concatenate([o[0] for o in outs], axis=0)
  col_emb = jnp.concatenate([o[1] for o in outs], axis=0)
  return (row_emb, col_emb)


# speedup vs baseline: 1.0746x; 1.0122x over previous
import functools

import jax
import jax.numpy as jnp
from jax import lax
from jax.experimental import pallas as pl
from jax.experimental.pallas import tpu as pltpu
from jax.experimental.pallas import tpu_sc as plsc

_B, _N, _S, _D = 8, 2048, 64, 128

_NC, _NS = 2, 16
_NW = _NC * _NS
_CH = 4


def _make_sc_body(nb, boff):
  rpw = (nb * _N) // _NW
  cpw = rpw // _CH
  wpb = _NW // nb
  ncolw = (nb * _S) // 16

  def body(dm_ref, idx_ref, rowg_ref, colg_ref,
           idx_v, idxc_v, cid_v, inbuf0, inbuf1, inbuf2, inbuf3,
           outbuf, colbuf, sem0, sem1, sem2, sem3, semc):
    c = lax.axis_index("c")
    s = lax.axis_index("s")
    wid = s * _NC + c
    b = wid // wpb
    row0 = boff * _N + wid * rpw

    pltpu.sync_copy(idx_ref.at[pl.ds((boff + b) * _S, _S)], idx_v)
    ivs = [idx_v[pl.ds(k * 16, 16)] for k in range(4)]

    @pl.when(wid < ncolw)
    def _():
      bc = wid // (_S // 16)
      pltpu.sync_copy(idx_ref.at[pl.ds((boff + bc) * _S, _S)], idxc_v)
      cid_v[...] = idxc_v[pl.ds((wid % (_S // 16)) * 16, 16)] + (
          (boff + bc) * _N)
      pltpu.async_copy(dm_ref.at[cid_v], colbuf, semc)

    bufs = (inbuf0, inbuf1, inbuf2, inbuf3)
    sems = (sem0, sem1, sem2, sem3)
    nbuf = 4

    def _start(g, t):
      pltpu.async_copy(dm_ref.at[pl.ds(row0 + g * _CH, _CH)], bufs[t],
                       sems[t])

    def _wait(g, t):
      pltpu.make_async_copy(dm_ref.at[pl.ds(row0 + g * _CH, _CH)],
                            bufs[t], sems[t]).wait()

    for g in range(nbuf - 1):
      _start(g, g)

    def chunk4(gg, carry):
      for t in range(nbuf):
        g = gg * nbuf + t

        @pl.when(g + nbuf - 1 < cpw)
        def _():
          _start(g + nbuf - 1, (t + nbuf - 1) % nbuf)

        _wait(g, t)
        for r in range(_CH):
          rvec = jnp.full((16,), r, jnp.int32)
          for k in range(4):
            outbuf[g * _CH + r, pl.ds(k * 16, 16)] = (
                plsc.load_gather(bufs[t], [rvec, ivs[k]]))
      return carry

    lax.fori_loop(0, cpw // nbuf, chunk4, 0)
    pltpu.sync_copy(outbuf, rowg_ref.at[pl.ds(wid * rpw, rpw)])

    @pl.when(wid < ncolw)
    def _():
      pltpu.make_async_copy(dm_ref.at[cid_v], colbuf, semc).wait()
      pltpu.sync_copy(colbuf, colg_ref.at[pl.ds(wid * 16, 16)])

  return body


def _sc_gather(dm2, idxflat, nb, boff):
  mesh = plsc.VectorSubcoreMesh(core_axis_name="c", subcore_axis_name="s",
                                num_cores=_NC, num_subcores=_NS)
  rpw = (nb * _N) // _NW
  f = pl.kernel(
      _make_sc_body(nb, boff),
      out_type=[
          jax.ShapeDtypeStruct((nb * _N, _S), jnp.float32),
          jax.ShapeDtypeStruct((nb * _S, _N), jnp.float32),
      ],
      mesh=mesh,
      scratch_types=[
          pltpu.VMEM((_S,), jnp.int32),
          pltpu.VMEM((_S,), jnp.int32),
          pltpu.VMEM((16,), jnp.int32),
          pltpu.VMEM((_CH, _N), jnp.float32),
          pltpu.VMEM((_CH, _N), jnp.float32),
          pltpu.VMEM((_CH, _N), jnp.float32),
          pltpu.VMEM((_CH, _N), jnp.float32),
          pltpu.VMEM((rpw, _S), jnp.float32),
          pltpu.VMEM((16, _N), jnp.float32),
          pltpu.SemaphoreType.DMA,
          pltpu.SemaphoreType.DMA,
          pltpu.SemaphoreType.DMA,
          pltpu.SemaphoreType.DMA,
          pltpu.SemaphoreType.DMA,
      ],
      compiler_params=pltpu.CompilerParams(needs_layout_passes=False),
      cost_estimate=pl.CostEstimate(
          flops=0,
          bytes_accessed=nb * _N * _N * 4 + nb * (_N + _S) * _S * 8,
          transcendentals=0,
      ),
  )
  return f(dm2, idxflat)


_RB = 2048
_PREC = lax.Precision.HIGHEST


def _xor_perm(j):
  r = lax.broadcasted_iota(jnp.int32, (_S, _S), 0)
  c = lax.broadcasted_iota(jnp.int32, (_S, _S), 1)
  return ((r ^ j) == c).astype(jnp.float32)


def _bitonic_sort_lanes(x, use_mxu):
  lane = lax.broadcasted_iota(jnp.int32, (1, _S), 1)
  k = 2
  while k <= _S:
    j = k // 2
    while j >= 1:
      lower = (lane & j) == 0
      if use_mxu:
        xp = lax.dot_general(x, _xor_perm(j), (((1,), (0,)), ((), ())),
                             preferred_element_type=jnp.float32)
      else:
        xp = jnp.where(lower,
                       pltpu.roll(x, _S - j, 1),
                       pltpu.roll(x, j, 1))
      take_min = lower == ((lane & k) == 0)
      x = jnp.where(take_min, jnp.minimum(x, xp), jnp.maximum(x, xp))
      j //= 2
    k *= 2
  return x


def _tc_body(rowg_ref, colg_ref, wr_ref, br_ref, wc_ref, bc_ref,
             row_out_ref, col_out_ref):
  cg = jnp.transpose(colg_ref[0], (1, 0))
  xs = _bitonic_sort_lanes(
      jnp.concatenate([rowg_ref[0], cg], axis=0), use_mxu=True)
  rs = xs[:_RB]
  cs = xs[_RB:]
  remb = lax.dot_general(rs, wr_ref[...], (((1,), (1,)), ((), ())),
                         preferred_element_type=jnp.float32,
                         precision=_PREC)
  row_out_ref[0] = remb + br_ref[...]
  cemb = lax.dot_general(cs, wc_ref[...], (((1,), (1,)), ((), ())),
                         preferred_element_type=jnp.float32,
                         precision=_PREC)
  col_out_ref[0] = cemb + bc_ref[...]


def _tc_sort_embed(rowg3, colg3, Wr, br2, Wc, bc2, Bv):
  grid = (Bv, _N // _RB)
  return pl.pallas_call(
      _tc_body,
      grid=grid,
      in_specs=[
          pl.BlockSpec((1, _RB, _S), lambda b, i: (b, i, 0)),
          pl.BlockSpec((1, _S, _RB), lambda b, i: (b, 0, i)),
          pl.BlockSpec((_D, _S), lambda b, i: (0, 0)),
          pl.BlockSpec((1, _D), lambda b, i: (0, 0)),
          pl.BlockSpec((_D, _S), lambda b, i: (0, 0)),
          pl.BlockSpec((1, _D), lambda b, i: (0, 0)),
      ],
      out_specs=[
          pl.BlockSpec((1, _RB, _D), lambda b, i: (b, i, 0)),
          pl.BlockSpec((1, _RB, _D), lambda b, i: (b, i, 0)),
      ],
      out_shape=[
          jax.ShapeDtypeStruct((Bv, _N, _D), jnp.float32),
          jax.ShapeDtypeStruct((Bv, _N, _D), jnp.float32),
      ],
      compiler_params=pltpu.CompilerParams(
          dimension_semantics=("arbitrary", "arbitrary"),
      ),
      cost_estimate=pl.CostEstimate(
          flops=2 * Bv * _N * _S * (21 * _S + 2 * _D),
          bytes_accessed=Bv * _N * (_S * 8 + _D * 8) * 4,
          transcendentals=0,
      ),
  )(rowg3, colg3, Wr, br2, Wc, bc2)


def kernel(distance_matrix, Wr, br, Wc, bc, phase):
  Bv = distance_matrix.shape[0]
  ikey = jax.random.key(42)
  ri = jax.random.randint(ikey, (8, 1, _S), 0, _N)
  idx = jnp.broadcast_to(ri[:, None, :, :], (8, Bv // 8, 1, _S))
  idxflat = idx.reshape(Bv * _S).astype(jnp.int32)

  dm2 = distance_matrix.reshape(Bv * _N, _N)
  br2 = br.reshape(1, _D)
  bc2 = bc.reshape(1, _D)

  nb = Bv // 4
  parts = []
  for boff in range(0, Bv, nb):
    rowg, colg = _sc_gather(dm2, idxflat, nb, boff)
    parts.append((rowg, colg))
  outs = []
  for rowg, colg in parts:
    outs.append(_tc_sort_embed(rowg.reshape(nb, _N, _S),
                               colg.reshape(nb, _S, _N),
                               Wr, br2, Wc, bc2, nb))
  row_emb = jnp.concatenate([o[0] for o in outs], axis=0)
  col_emb = jnp.concatenate([o[1] for o in outs], axis=0)
  return (row_emb, col_emb)
